# baseline (speedup 1.0000x reference)
import functools

import jax
import jax.numpy as jnp
import numpy as np
from jax import lax
from jax.experimental import pallas as pl
from jax.experimental.pallas import tpu as pltpu
from jax.experimental.pallas import tpu_sc as plsc

_N = 174592
_NC = 16
_K = max(int(_N * 0.01), 2)
_W = 16
_R = _N // _W
_CHUNK = 496
_NCH = _R // _CHUNK
_GPC = _CHUNK // 16
_VPW = _R // 16
_MINI32 = np.int32(-2147483648)
_M31 = np.int32(0x7FFFFFFF)


def _sig(v):
    return 1.0 / (1.0 + jnp.exp(-v))


def _mono_key(bits):
    return bits ^ (lax.shift_right_arithmetic(bits, 31) & _M31)


def _sc_body(x_ref, c_ref, pos_ref, neg_ref, joint_ref, scal_ref,
             inbuf, inbuf2, dsem0, dsem1, cbuf, ubuf, sbuf, jbuf, pmbuf, nmbuf,
             histp, histn, lredp, lredn, mbufp, mbufn, totp, totn,
             svec, fvec, scalb, sumb, fgb,
             shp, shn, shsum, shfg):
    wid = lax.axis_index("s")
    base = wid * _R
    lane = lax.iota(jnp.int32, 16)
    zi16 = jnp.zeros((16,), jnp.int32)
    zf16 = jnp.zeros((16,), jnp.float32)
    oi16 = jnp.ones((16,), jnp.int32)

    pltpu.sync_copy(c_ref.at[pl.ds(base, _R)], cbuf)

    def zb0(i, c):
        histp[pl.ds(i * 16, 16)] = zi16
        return c

    lax.fori_loop(0, 256, zb0, 0, unroll=4)

    def xsrc(ci):
        return x_ref.at[pl.ds(base + ci * _CHUNK, _CHUNK), :]

    def process(buf, ci, a):
        def group_body(g, a):
            rows = g * 16 + lane
            cols = [plsc.load_gather(buf, [rows, zi16])]
            for j in range(1, _NC):
                cj = jnp.full((16,), j, jnp.int32)
                cols.append(plsc.load_gather(buf, [rows, cj]))
            while len(cols) > 1:
                cols = [jnp.maximum(cols[i], cols[i + 1])
                        for i in range(0, len(cols), 2)]
            m = cols[0]
            off = ci * _CHUNK + g * 16
            key = _mono_key(plsc.bitcast(m, jnp.int32))
            ubuf[pl.ds(off, 16)] = key
            byte = lax.shift_right_logical(key ^ _MINI32, 24)
            plsc.addupdate_scatter(histp, [byte * 16 + lane], oi16)
            s = _sig(m)
            sbuf[pl.ds(off, 16)] = s
            jbuf[pl.ds(off, 16)] = s * _sig(cbuf[pl.ds(off, 16)])
            return a + s

        return lax.fori_loop(0, _GPC, group_body, a)

    pltpu.async_copy(xsrc(0), inbuf, dsem0)

    def chunk2_body(i2, acc):
        c0 = 2 * i2
        c1 = 2 * i2 + 1
        pltpu.make_async_copy(xsrc(c0), inbuf, dsem0).wait()
        pltpu.async_copy(xsrc(c1), inbuf2, dsem1)
        acc = process(inbuf, c0, acc)
        pltpu.make_async_copy(xsrc(c1), inbuf2, dsem1).wait()

        @pl.when(c1 + 1 < _NCH)
        def _():
            pltpu.async_copy(xsrc(c1 + 1), inbuf, dsem0)

        return process(inbuf2, c1, acc)

    acc_s = lax.fori_loop(0, _NCH // 2, chunk2_body, zf16)
    pltpu.sync_copy(jbuf, joint_ref.at[pl.ds(base, _R)])

    kp = jnp.int32(_K)
    kn = jnp.int32(_K)
    prefp = jnp.int32(0)
    prefn = jnp.int32(0)
    selp = jnp.int32(0)
    seln = jnp.int32(0)

    for r in range(4):
        shift = 24 - 8 * r

        if r > 0:
            def zb(i, c):
                histp[pl.ds(i * 16, 16)] = zi16
                histn[pl.ds(i * 16, 16)] = zi16
                return c

            lax.fori_loop(0, 256, zb, 0, unroll=4)

        if r == 0:
            pass
        else:
            hs = shift + 8
            pp, pn = prefp, prefn

            def sbr(g, c):
                ub = ubuf[pl.ds(g * 16, 16)] ^ _MINI32
                hi = lax.shift_right_logical(ub, hs)
                byte = lax.shift_right_logical(ub, shift) & 255
                fidx = byte * 16 + lane
                plsc.addupdate_scatter(histp, [fidx], oi16, mask=(hi == pp))
                plsc.addupdate_scatter(histn, [fidx], oi16, mask=(hi == pn))
                return c

            lax.fori_loop(0, _VPW, sbr, 0, unroll=2)

        hn_r = histp if r == 0 else histn

        def trp(bg, c):
            flat = bg * 256 + lane * 16
            tps = [plsc.load_gather(histp, [flat + j]) for j in range(16)]
            tns = [plsc.load_gather(hn_r, [flat + j]) for j in range(16)]
            while len(tps) > 1:
                tps = [tps[i] + tps[i + 1] for i in range(0, len(tps), 2)]
                tns = [tns[i] + tns[i + 1] for i in range(0, len(tns), 2)]
            lredp[pl.ds(bg * 16, 16)] = tps[0]
            lredn[pl.ds(bg * 16, 16)] = tns[0]
            return c

        lax.fori_loop(0, 16, trp, 0)

        pltpu.sync_copy(lredp, shp.at[pl.ds(wid * 256, 256)])
        pltpu.sync_copy(lredn, shn.at[pl.ds(wid * 256, 256)])
        plsc.subcore_barrier()
        pltpu.sync_copy(shp, mbufp)
        pltpu.sync_copy(shn, mbufn)
        plsc.subcore_barrier()

        def mg(bg, c):
            def mwp(w, a):
                return a + mbufp[pl.ds(w * 256 + bg * 16, 16)]

            def mwn(w, a):
                return a + mbufn[pl.ds(w * 256 + bg * 16, 16)]

            totp[pl.ds(bg * 16, 16)] = lax.fori_loop(0, _W, mwp, zi16)
            totn[pl.ds(bg * 16, 16)] = lax.fori_loop(0, _W, mwn, zi16)
            return c

        lax.fori_loop(0, 16, mg, 0)

        def group_sums(totref):
            t = plsc.load_gather(totref, [lane * 16])
            for j in range(1, 16):
                t = t + plsc.load_gather(totref, [lane * 16 + j])
            return t

        def sel_top(totref, k):
            gsum = group_sums(totref)
            csum = plsc.cumsum(gsum)
            tot = jnp.sum(gsum)
            suf = tot - csum + gsum
            gp = plsc.all_reduce_population_count(suf >= k)[0] - 1
            above_g = jnp.sum(jnp.where(lane == gp, suf - gsum, 0))
            rgrp = lax.rev(totref[pl.ds(gp * 16, 16)], (0,))
            rcs = plsc.cumsum(rgrp)
            cond = (above_g + rcs) >= k
            t0 = 16 - plsc.all_reduce_population_count(cond)[0]
            sel = gp * 16 + 15 - t0
            acc = (above_g
                   + jnp.sum(jnp.where(lane == t0, rcs, 0))
                   - jnp.sum(jnp.where(lane == t0, rgrp, 0)))
            return sel, acc

        def sel_bot(totref, k):
            gsum = group_sums(totref)
            csum = plsc.cumsum(gsum)
            gn = 16 - plsc.all_reduce_population_count(csum >= k)[0]
            below_g = (jnp.sum(jnp.where(lane == gn, csum, 0))
                       - jnp.sum(jnp.where(lane == gn, gsum, 0)))
            grp = totref[pl.ds(gn * 16, 16)]
            cs2 = plsc.cumsum(grp)
            cond = (below_g + cs2) >= k
            t0 = 16 - plsc.all_reduce_population_count(cond)[0]
            sel = gn * 16 + t0
            acc = (below_g
                   + jnp.sum(jnp.where(lane == t0, cs2, 0))
                   - jnp.sum(jnp.where(lane == t0, grp, 0)))
            return sel, acc

        selp, accp = sel_top(totp, kp)
        kp = kp - accp
        prefp = lax.shift_left(prefp, 8) | selp

        seln, accn = sel_bot(totn, kn)
        kn = kn - accn
        prefn = lax.shift_left(prefn, 8) | seln

    Tp = prefp ^ _MINI32
    Tn = prefn ^ _MINI32

    vp = plsc.load_gather(mbufp, [lane * 256 + selp])
    prep = jnp.sum(jnp.where(lane < wid, vp, 0))
    locp = jnp.sum(jnp.where(lane == wid, vp, 0))
    budp = jnp.maximum(jnp.int32(0), jnp.minimum(kp - prep, locp))

    vn = plsc.load_gather(mbufn, [lane * 256 + seln])
    pren = jnp.sum(jnp.where(lane < wid, vn, 0))
    locn = jnp.sum(jnp.where(lane == wid, vn, 0))
    budn = jnp.maximum(jnp.int32(0), jnp.minimum(kn - pren, locn))

    def mb(g, fga):
        key = ubuf[pl.ds(g * 16, 16)]
        s = sbuf[pl.ds(g * 16, 16)]
        gt = key > Tp
        lt = key < Tn
        pmbuf[pl.ds(g * 16, 16)] = jnp.where(gt, jnp.int32(1), jnp.int32(0))
        nmbuf[pl.ds(g * 16, 16)] = jnp.where(lt, jnp.int32(1), jnp.int32(0))
        return fga + jnp.where(gt, s, jnp.float32(0.0))

    fgacc = lax.fori_loop(0, _VPW, mb, zf16, unroll=2)

    @pl.when(budp > 0)
    def _tie_pos():
        def tb(g, rem):
            key = ubuf[pl.ds(g * 16, 16)]
            eq = key == Tp
            eqi = jnp.where(eq, jnp.int32(1), jnp.int32(0))
            cs = plsc.cumsum(eqi)
            mark = eq & (cs <= rem)
            pm = pmbuf[pl.ds(g * 16, 16)]
            pmbuf[pl.ds(g * 16, 16)] = jnp.where(mark, jnp.int32(1), pm)
            return rem - jnp.sum(eqi)

        lax.fori_loop(0, _VPW, tb, budp)

    @pl.when(budn > 0)
    def _tie_neg():
        def tb(g, rem):
            key = ubuf[pl.ds(g * 16, 16)]
            eq = key == Tn
            eqi = jnp.where(eq, jnp.int32(1), jnp.int32(0))
            cs = plsc.cumsum(eqi)
            mark = eq & (cs <= rem)
            nm = nmbuf[pl.ds(g * 16, 16)]
            nmbuf[pl.ds(g * 16, 16)] = jnp.where(mark, jnp.int32(1), nm)
            return rem - jnp.sum(eqi)

        lax.fori_loop(0, _VPW, tb, budn)

    @pl.when(Tp == Tn)
    def _fix_overlap():
        def fx(g, c):
            pm = pmbuf[pl.ds(g * 16, 16)]
            nm = nmbuf[pl.ds(g * 16, 16)]
            pmbuf[pl.ds(g * 16, 16)] = jnp.where(nm > 0, jnp.int32(0), pm)
            return c

        lax.fori_loop(0, _VPW, fx, 0)

    pltpu.sync_copy(pmbuf, pos_ref.at[pl.ds(base, _R)])
    pltpu.sync_copy(nmbuf, neg_ref.at[pl.ds(base, _R)])
    svec[pl.ds(0, 16)] = acc_s
    fvec[pl.ds(0, 16)] = fgacc
    pltpu.sync_copy(svec, shsum.at[pl.ds(wid * 16, 16)])
    pltpu.sync_copy(fvec, shfg.at[pl.ds(wid * 16, 16)])
    plsc.subcore_barrier()

    @pl.when(wid == 0)
    def _scalars():
        pltpu.sync_copy(shsum, sumb)
        pltpu.sync_copy(shfg, fgb)

        def rs(w, a):
            return a + sumb[pl.ds(w * 16, 16)]

        def rf(w, a):
            return a + fgb[pl.ds(w * 16, 16)]

        sv = lax.fori_loop(0, _W, rs, zf16)
        fv = lax.fori_loop(0, _W, rf, zf16)
        s_dps = jnp.sum(sv) * jnp.float32(1.0 / _N)
        tvec = jnp.full((16,), Tp, jnp.int32)
        s_thr = jnp.max(_sig(plsc.bitcast(_mono_key(tvec), jnp.float32)))
        fg = jnp.sum(fv) + kp.astype(jnp.float32) * s_thr
        out = jnp.where(lane == 0, jnp.full((16,), s_dps),
                        jnp.where(lane == 1, jnp.full((16,), fg), zf16))
        scalb[pl.ds(0, 16)] = out
        pltpu.sync_copy(scalb, scal_ref)


_sc_call = functools.partial(
    pl.kernel,
    out_type=[
        jax.ShapeDtypeStruct((_N,), jnp.int32),
        jax.ShapeDtypeStruct((_N,), jnp.int32),
        jax.ShapeDtypeStruct((_N,), jnp.float32),
        jax.ShapeDtypeStruct((16,), jnp.float32),
    ],
    mesh=plsc.VectorSubcoreMesh(
        core_axis_name="c", subcore_axis_name="s",
        num_cores=1, num_subcores=_W),
    compiler_params=pltpu.CompilerParams(
        needs_layout_passes=False, use_tc_tiling_on_sc=False),
    scratch_types=[
        pltpu.VMEM((_CHUNK, _NC), jnp.float32),
        pltpu.VMEM((_CHUNK, _NC), jnp.float32),
        pltpu.SemaphoreType.DMA,
        pltpu.SemaphoreType.DMA,
        pltpu.VMEM((_R,), jnp.float32),
        pltpu.VMEM((_R,), jnp.int32),
        pltpu.VMEM((_R,), jnp.float32),
        pltpu.VMEM((_R,), jnp.float32),
        pltpu.VMEM((_R,), jnp.int32),
        pltpu.VMEM((_R,), jnp.int32),
        pltpu.VMEM((4096,), jnp.int32),
        pltpu.VMEM((4096,), jnp.int32),
        pltpu.VMEM((256,), jnp.int32),
        pltpu.VMEM((256,), jnp.int32),
        pltpu.VMEM((_W * 256,), jnp.int32),
        pltpu.VMEM((_W * 256,), jnp.int32),
        pltpu.VMEM((256,), jnp.int32),
        pltpu.VMEM((256,), jnp.int32),
        pltpu.VMEM((16,), jnp.float32),
        pltpu.VMEM((16,), jnp.float32),
        pltpu.VMEM((16,), jnp.float32),
        pltpu.VMEM((_W * 16,), jnp.float32),
        pltpu.VMEM((_W * 16,), jnp.float32),
        pltpu.VMEM_SHARED((_W * 256,), jnp.int32),
        pltpu.VMEM_SHARED((_W * 256,), jnp.int32),
        pltpu.VMEM_SHARED((_W * 16,), jnp.float32),
        pltpu.VMEM_SHARED((_W * 16,), jnp.float32),
    ],
)(_sc_body)


def kernel(t_cls_scores, t_centernesses):
    cent = t_centernesses.reshape(-1)
    pos_i, neg_i, joint, scal = _sc_call(t_cls_scores, cent)
    return (pos_i.astype(jnp.bool_), neg_i.astype(jnp.bool_), joint,
            scal[1], scal[0], joint)


# speedup vs baseline: 1.0074x; 1.0074x over previous
import functools

import jax
import jax.numpy as jnp
import numpy as np
from jax import lax
from jax.experimental import pallas as pl
from jax.experimental.pallas import tpu as pltpu
from jax.experimental.pallas import tpu_sc as plsc

_N = 174592
_NC = 16
_K = max(int(_N * 0.01), 2)
_W = 16
_R = _N // _W
_CHUNK = 496
_NCH = _R // _CHUNK
_GPC = _CHUNK // 16
_VPW = _R // 16
_MINI32 = np.int32(-2147483648)
_M31 = np.int32(0x7FFFFFFF)


def _sig(v):
    return 1.0 / (1.0 + jnp.exp(-v))


def _mono_key(bits):
    return bits ^ (lax.shift_right_arithmetic(bits, 31) & _M31)


def _sc_body(x_ref, c_ref, pos_ref, neg_ref, joint_ref, scal_ref,
             inbuf, inbuf2, dsem0, dsem1, cbuf, ubuf, sbuf, jbuf, pmbuf, nmbuf,
             histp, histn, lredp, lredn, mbufp, mbufn, totp, totn,
             svec, fvec, scalb, sumb, fgb,
             shp, shn, shsum, shfg):
    wid = lax.axis_index("s")
    base = wid * _R
    lane = lax.iota(jnp.int32, 16)
    zi16 = jnp.zeros((16,), jnp.int32)
    zf16 = jnp.zeros((16,), jnp.float32)
    oi16 = jnp.ones((16,), jnp.int32)

    pltpu.sync_copy(c_ref.at[pl.ds(base, _R)], cbuf)

    def zb0(i, c):
        histp[pl.ds(i * 16, 16)] = zi16
        return c

    lax.fori_loop(0, 256, zb0, 0, unroll=4)

    def xsrc(ci):
        return x_ref.at[pl.ds(base + ci * _CHUNK, _CHUNK), :]

    def process(buf, ci, a):
        def group_body(g, a):
            rows = g * 16 + lane
            m = plsc.load_gather(buf, [rows, zi16])
            for j in range(1, _NC):
                cj = jnp.full((16,), j, jnp.int32)
                m = jnp.maximum(m, plsc.load_gather(buf, [rows, cj]))
            off = ci * _CHUNK + g * 16
            key = _mono_key(plsc.bitcast(m, jnp.int32))
            ubuf[pl.ds(off, 16)] = key
            byte = lax.shift_right_logical(key ^ _MINI32, 24)
            plsc.addupdate_scatter(histp, [byte * 16 + lane], oi16)
            s = _sig(m)
            sbuf[pl.ds(off, 16)] = s
            jbuf[pl.ds(off, 16)] = s * _sig(cbuf[pl.ds(off, 16)])
            return a + s

        return lax.fori_loop(0, _GPC, group_body, a)

    pltpu.async_copy(xsrc(0), inbuf, dsem0)

    def chunk2_body(i2, acc):
        c0 = 2 * i2
        c1 = 2 * i2 + 1
        pltpu.make_async_copy(xsrc(c0), inbuf, dsem0).wait()
        pltpu.async_copy(xsrc(c1), inbuf2, dsem1)
        acc = process(inbuf, c0, acc)
        pltpu.make_async_copy(xsrc(c1), inbuf2, dsem1).wait()

        @pl.when(c1 + 1 < _NCH)
        def _():
            pltpu.async_copy(xsrc(c1 + 1), inbuf, dsem0)

        return process(inbuf2, c1, acc)

    acc_s = lax.fori_loop(0, _NCH // 2, chunk2_body, zf16)
    pltpu.sync_copy(jbuf, joint_ref.at[pl.ds(base, _R)])

    kp = jnp.int32(_K)
    kn = jnp.int32(_K)
    prefp = jnp.int32(0)
    prefn = jnp.int32(0)
    selp = jnp.int32(0)
    seln = jnp.int32(0)

    for r in range(4):
        shift = 24 - 8 * r

        if r > 0:
            def zb(i, c):
                histp[pl.ds(i * 16, 16)] = zi16
                histn[pl.ds(i * 16, 16)] = zi16
                return c

            lax.fori_loop(0, 256, zb, 0, unroll=4)

        if r == 0:
            pass
        else:
            hs = shift + 8
            pp, pn = prefp, prefn

            def sbr(g, c):
                ub = ubuf[pl.ds(g * 16, 16)] ^ _MINI32
                hi = lax.shift_right_logical(ub, hs)
                byte = lax.shift_right_logical(ub, shift) & 255
                fidx = byte * 16 + lane
                plsc.addupdate_scatter(histp, [fidx], oi16, mask=(hi == pp))
                plsc.addupdate_scatter(histn, [fidx], oi16, mask=(hi == pn))
                return c

            lax.fori_loop(0, _VPW, sbr, 0, unroll=2)

        hn_r = histp if r == 0 else histn

        def trp(bg, c):
            flat = bg * 256 + lane * 16
            tp = plsc.load_gather(histp, [flat])
            tn = plsc.load_gather(hn_r, [flat])
            for j in range(1, 16):
                tp = tp + plsc.load_gather(histp, [flat + j])
                tn = tn + plsc.load_gather(hn_r, [flat + j])
            lredp[pl.ds(bg * 16, 16)] = tp
            lredn[pl.ds(bg * 16, 16)] = tn
            return c

        lax.fori_loop(0, 16, trp, 0)

        pltpu.sync_copy(lredp, shp.at[pl.ds(wid * 256, 256)])
        pltpu.sync_copy(lredn, shn.at[pl.ds(wid * 256, 256)])
        plsc.subcore_barrier()
        pltpu.sync_copy(shp, mbufp)
        pltpu.sync_copy(shn, mbufn)
        plsc.subcore_barrier()

        def mg(bg, c):
            def mwp(w, a):
                return a + mbufp[pl.ds(w * 256 + bg * 16, 16)]

            def mwn(w, a):
                return a + mbufn[pl.ds(w * 256 + bg * 16, 16)]

            totp[pl.ds(bg * 16, 16)] = lax.fori_loop(0, _W, mwp, zi16)
            totn[pl.ds(bg * 16, 16)] = lax.fori_loop(0, _W, mwn, zi16)
            return c

        lax.fori_loop(0, 16, mg, 0)

        def group_sums(totref):
            t = plsc.load_gather(totref, [lane * 16])
            for j in range(1, 16):
                t = t + plsc.load_gather(totref, [lane * 16 + j])
            return t

        def sel_top(totref, k):
            gsum = group_sums(totref)
            csum = plsc.cumsum(gsum)
            tot = jnp.sum(gsum)
            suf = tot - csum + gsum
            gp = plsc.all_reduce_population_count(suf >= k)[0] - 1
            above_g = jnp.sum(jnp.where(lane == gp, suf - gsum, 0))
            rgrp = lax.rev(totref[pl.ds(gp * 16, 16)], (0,))
            rcs = plsc.cumsum(rgrp)
            cond = (above_g + rcs) >= k
            t0 = 16 - plsc.all_reduce_population_count(cond)[0]
            sel = gp * 16 + 15 - t0
            acc = (above_g
                   + jnp.sum(jnp.where(lane == t0, rcs, 0))
                   - jnp.sum(jnp.where(lane == t0, rgrp, 0)))
            return sel, acc

        def sel_bot(totref, k):
            gsum = group_sums(totref)
            csum = plsc.cumsum(gsum)
            gn = 16 - plsc.all_reduce_population_count(csum >= k)[0]
            below_g = (jnp.sum(jnp.where(lane == gn, csum, 0))
                       - jnp.sum(jnp.where(lane == gn, gsum, 0)))
            grp = totref[pl.ds(gn * 16, 16)]
            cs2 = plsc.cumsum(grp)
            cond = (below_g + cs2) >= k
            t0 = 16 - plsc.all_reduce_population_count(cond)[0]
            sel = gn * 16 + t0
            acc = (below_g
                   + jnp.sum(jnp.where(lane == t0, cs2, 0))
                   - jnp.sum(jnp.where(lane == t0, grp, 0)))
            return sel, acc

        selp, accp = sel_top(totp, kp)
        kp = kp - accp
        prefp = lax.shift_left(prefp, 8) | selp

        seln, accn = sel_bot(totn, kn)
        kn = kn - accn
        prefn = lax.shift_left(prefn, 8) | seln

    Tp = prefp ^ _MINI32
    Tn = prefn ^ _MINI32

    vp = plsc.load_gather(mbufp, [lane * 256 + selp])
    prep = jnp.sum(jnp.where(lane < wid, vp, 0))
    locp = jnp.sum(jnp.where(lane == wid, vp, 0))
    budp = jnp.maximum(jnp.int32(0), jnp.minimum(kp - prep, locp))

    vn = plsc.load_gather(mbufn, [lane * 256 + seln])
    pren = jnp.sum(jnp.where(lane < wid, vn, 0))
    locn = jnp.sum(jnp.where(lane == wid, vn, 0))
    budn = jnp.maximum(jnp.int32(0), jnp.minimum(kn - pren, locn))

    def mb(g, fga):
        key = ubuf[pl.ds(g * 16, 16)]
        s = sbuf[pl.ds(g * 16, 16)]
        gt = key > Tp
        lt = key < Tn
        pmbuf[pl.ds(g * 16, 16)] = jnp.where(gt, jnp.int32(1), jnp.int32(0))
        nmbuf[pl.ds(g * 16, 16)] = jnp.where(lt, jnp.int32(1), jnp.int32(0))
        return fga + jnp.where(gt, s, jnp.float32(0.0))

    fgacc = lax.fori_loop(0, _VPW, mb, zf16, unroll=2)

    @pl.when(budp > 0)
    def _tie_pos():
        def tb(g, rem):
            key = ubuf[pl.ds(g * 16, 16)]
            eq = key == Tp
            eqi = jnp.where(eq, jnp.int32(1), jnp.int32(0))
            cs = plsc.cumsum(eqi)
            mark = eq & (cs <= rem)
            pm = pmbuf[pl.ds(g * 16, 16)]
            pmbuf[pl.ds(g * 16, 16)] = jnp.where(mark, jnp.int32(1), pm)
            return rem - jnp.sum(eqi)

        lax.fori_loop(0, _VPW, tb, budp)

    @pl.when(budn > 0)
    def _tie_neg():
        def tb(g, rem):
            key = ubuf[pl.ds(g * 16, 16)]
            eq = key == Tn
            eqi = jnp.where(eq, jnp.int32(1), jnp.int32(0))
            cs = plsc.cumsum(eqi)
            mark = eq & (cs <= rem)
            nm = nmbuf[pl.ds(g * 16, 16)]
            nmbuf[pl.ds(g * 16, 16)] = jnp.where(mark, jnp.int32(1), nm)
            return rem - jnp.sum(eqi)

        lax.fori_loop(0, _VPW, tb, budn)

    @pl.when(Tp == Tn)
    def _fix_overlap():
        def fx(g, c):
            pm = pmbuf[pl.ds(g * 16, 16)]
            nm = nmbuf[pl.ds(g * 16, 16)]
            pmbuf[pl.ds(g * 16, 16)] = jnp.where(nm > 0, jnp.int32(0), pm)
            return c

        lax.fori_loop(0, _VPW, fx, 0)

    pltpu.sync_copy(pmbuf, pos_ref.at[pl.ds(base, _R)])
    pltpu.sync_copy(nmbuf, neg_ref.at[pl.ds(base, _R)])
    svec[pl.ds(0, 16)] = acc_s
    fvec[pl.ds(0, 16)] = fgacc
    pltpu.sync_copy(svec, shsum.at[pl.ds(wid * 16, 16)])
    pltpu.sync_copy(fvec, shfg.at[pl.ds(wid * 16, 16)])
    plsc.subcore_barrier()

    @pl.when(wid == 0)
    def _scalars():
        pltpu.sync_copy(shsum, sumb)
        pltpu.sync_copy(shfg, fgb)

        def rs(w, a):
            return a + sumb[pl.ds(w * 16, 16)]

        def rf(w, a):
            return a + fgb[pl.ds(w * 16, 16)]

        sv = lax.fori_loop(0, _W, rs, zf16)
        fv = lax.fori_loop(0, _W, rf, zf16)
        s_dps = jnp.sum(sv) * jnp.float32(1.0 / _N)
        tvec = jnp.full((16,), Tp, jnp.int32)
        s_thr = jnp.max(_sig(plsc.bitcast(_mono_key(tvec), jnp.float32)))
        fg = jnp.sum(fv) + kp.astype(jnp.float32) * s_thr
        out = jnp.where(lane == 0, jnp.full((16,), s_dps),
                        jnp.where(lane == 1, jnp.full((16,), fg), zf16))
        scalb[pl.ds(0, 16)] = out
        pltpu.sync_copy(scalb, scal_ref)


_sc_call = functools.partial(
    pl.kernel,
    out_type=[
        jax.ShapeDtypeStruct((_N,), jnp.int32),
        jax.ShapeDtypeStruct((_N,), jnp.int32),
        jax.ShapeDtypeStruct((_N,), jnp.float32),
        jax.ShapeDtypeStruct((16,), jnp.float32),
    ],
    mesh=plsc.VectorSubcoreMesh(
        core_axis_name="c", subcore_axis_name="s",
        num_cores=1, num_subcores=_W),
    compiler_params=pltpu.CompilerParams(
        needs_layout_passes=False, use_tc_tiling_on_sc=False),
    scratch_types=[
        pltpu.VMEM((_CHUNK, _NC), jnp.float32),
        pltpu.VMEM((_CHUNK, _NC), jnp.float32),
        pltpu.SemaphoreType.DMA,
        pltpu.SemaphoreType.DMA,
        pltpu.VMEM((_R,), jnp.float32),
        pltpu.VMEM((_R,), jnp.int32),
        pltpu.VMEM((_R,), jnp.float32),
        pltpu.VMEM((_R,), jnp.float32),
        pltpu.VMEM((_R,), jnp.int32),
        pltpu.VMEM((_R,), jnp.int32),
        pltpu.VMEM((4096,), jnp.int32),
        pltpu.VMEM((4096,), jnp.int32),
        pltpu.VMEM((256,), jnp.int32),
        pltpu.VMEM((256,), jnp.int32),
        pltpu.VMEM((_W * 256,), jnp.int32),
        pltpu.VMEM((_W * 256,), jnp.int32),
        pltpu.VMEM((256,), jnp.int32),
        pltpu.VMEM((256,), jnp.int32),
        pltpu.VMEM((16,), jnp.float32),
        pltpu.VMEM((16,), jnp.float32),
        pltpu.VMEM((16,), jnp.float32),
        pltpu.VMEM((_W * 16,), jnp.float32),
        pltpu.VMEM((_W * 16,), jnp.float32),
        pltpu.VMEM_SHARED((_W * 256,), jnp.int32),
        pltpu.VMEM_SHARED((_W * 256,), jnp.int32),
        pltpu.VMEM_SHARED((_W * 16,), jnp.float32),
        pltpu.VMEM_SHARED((_W * 16,), jnp.float32),
    ],
)(_sc_body)


def kernel(t_cls_scores, t_centernesses):
    cent = t_centernesses.reshape(-1)
    pos_i, neg_i, joint, scal = _sc_call(t_cls_scores, cent)
    return (pos_i.astype(jnp.bool_), neg_i.astype(jnp.bool_), joint,
            scal[1], scal[0], joint)
